# Initial kernel scaffold; baseline (speedup 1.0000x reference)
import jax
import jax.numpy as jnp
from jax.experimental import pallas as pl


def kernel(x, edge_index, batch, batch_size, neighbor_mask_node, neighbor_mask_edge, W_proj, b_proj, Wl0, bl0, Wr0, Wl1, bl1, Wr1, ln0_g, ln0_b, Wih0f, Whh0f, bih0f, bhh0f, Wih0b, Whh0b, bih0b, bhh0b, Wih1f, Whh1f, bih1f, bhh1f, Wih1b, Whh1b, bih1b, bhh1b, rnn_g, rnn_b, Wm1, bm1, mlp_g, mlp_b, Wm2, bm2, Wcat, bcat):
    raise NotImplementedError("write your pallas kernel here")


# speedup vs baseline: 18.3890x; 18.3890x over previous
import functools

import jax
import jax.numpy as jnp
from jax import lax
from jax.experimental import pallas as pl
from jax.experimental.pallas import tpu as pltpu
from jax.experimental.pallas import tpu_sc as plsc

NSEQ = 64
BS = 2048
N_NODES = 10000
D = 128
HGRU = 64
E_EDGES = 320000

NC, NS = 2, 16
NW = NC * NS
HD = D // 2
EPS = 20480
ECH = 128
NCHUNK = EPS // ECH
ACC_ROWS = 10240
ROWS_PW = ACC_ROWS // NS
PACK_ROWS = BS * NSEQ

_mesh = plsc.VectorSubcoreMesh(core_axis_name="c", subcore_axis_name="s")


def _wid():
    return lax.axis_index("s") * NC + lax.axis_index("c")


@functools.partial(
    pl.kernel,
    mesh=_mesh,
    out_type=jax.ShapeDtypeStruct((PACK_ROWS, D), jnp.float32),
    scratch_types=[
        pltpu.VMEM((BS // NW,), jnp.int32),
        pltpu.VMEM((BS // NW, D), jnp.float32),
        pltpu.SemaphoreType.DMA,
    ],
)
def _sc_pack(x_hbm, ridx_hbm, out_hbm, idx_v, rows_v, sem):
    w = _wid()
    base = w * (BS // NW)
    pltpu.sync_copy(ridx_hbm.at[pl.ds(base, BS // NW)], idx_v)
    pltpu.sync_copy(x_hbm.at[pl.ds(base, BS // NW)], rows_v)
    pltpu.async_copy(rows_v, out_hbm.at[idx_v], sem).wait()


@functools.partial(
    pl.kernel,
    mesh=_mesh,
    out_type=(
        jax.ShapeDtypeStruct((BS, HGRU), jnp.float32),
        jax.ShapeDtypeStruct((BS, HGRU), jnp.float32),
    ),
    scratch_types=[
        pltpu.VMEM((BS // NW,), jnp.int32),
        pltpu.VMEM((BS // NW, HGRU), jnp.float32),
        pltpu.VMEM((BS // NW, HGRU), jnp.float32),
        pltpu.SemaphoreType.DMA,
        pltpu.SemaphoreType.DMA,
    ],
    compiler_params=pltpu.CompilerParams(use_tc_tiling_on_sc=False),
)
def _sc_unpack(yf_hbm, yb_hbm, gidx_hbm, of_hbm, ob_hbm, idx_v, gf_v, gb_v,
               semf, semb):
    w = _wid()
    k = BS // NW
    base = w * k
    pltpu.sync_copy(gidx_hbm.at[pl.ds(base, k)], idx_v)
    cf = pltpu.async_copy(yf_hbm.at[idx_v], gf_v, semf)
    cb = pltpu.async_copy(yb_hbm.at[idx_v], gb_v, semb)
    cf.wait()
    cb.wait()
    pltpu.sync_copy(gf_v, of_hbm.at[pl.ds(base, k)])
    pltpu.sync_copy(gb_v, ob_hbm.at[pl.ds(base, k)])


def _edge_body(with_cnt, h_hbm, src_hbm, dst_hbm, zrow_hbm, zc_hbm, ones_hbm,
               acc_out, cnt_out, sidx_v, didx_v, rows_v, ones_v, acc_sh,
               cnt_sh, gsem):
    c = lax.axis_index("c")
    s = lax.axis_index("s")

    pltpu.sync_copy(zrow_hbm, rows_v)
    if with_cnt:
        pltpu.sync_copy(zc_hbm, ones_v)
    for j in range(ROWS_PW // ECH):
        pltpu.sync_copy(rows_v, acc_sh.at[pl.ds(s * ROWS_PW + j * ECH, ECH)])
        if with_cnt:
            pltpu.sync_copy(ones_v,
                            cnt_sh.at[pl.ds(s * ROWS_PW + j * ECH, ECH)])
    if with_cnt:
        pltpu.sync_copy(ones_hbm, ones_v)
    plsc.subcore_barrier()

    ebase = s * EPS
    srow = c * N_NODES

    def chunk(j, carry):
        off = pl.multiple_of(ebase + j * ECH, 8)
        pltpu.sync_copy(src_hbm.at[pl.ds(off, ECH)], sidx_v)
        pltpu.sync_copy(dst_hbm.at[pl.ds(off, ECH)], didx_v)
        for k in range(ECH // 16):
            sidx_v[pl.ds(k * 16, 16)] = sidx_v[pl.ds(k * 16, 16)] + srow
        pltpu.async_copy(h_hbm.at[sidx_v], rows_v, gsem).wait()
        pltpu.sync_copy(rows_v, acc_sh.at[didx_v], add=True)
        if with_cnt:
            pltpu.sync_copy(ones_v, cnt_sh.at[didx_v], add=True)
        return carry

    lax.fori_loop(0, NCHUNK, chunk, 0)
    plsc.subcore_barrier()

    rbase = s * ROWS_PW
    pltpu.sync_copy(acc_sh.at[pl.ds(rbase, ROWS_PW)],
                    acc_out.at[c, pl.ds(rbase, ROWS_PW)])
    if with_cnt:
        pltpu.sync_copy(cnt_sh.at[pl.ds(rbase, ROWS_PW)],
                        cnt_out.at[c, pl.ds(rbase, ROWS_PW)])


@functools.partial(
    pl.kernel,
    mesh=_mesh,
    out_type=(
        jax.ShapeDtypeStruct((NC, ACC_ROWS, HD), jnp.float32),
        jax.ShapeDtypeStruct((NC, ACC_ROWS, 16), jnp.float32),
    ),
    scratch_types=[
        pltpu.VMEM((ECH,), jnp.int32),
        pltpu.VMEM((ECH,), jnp.int32),
        pltpu.VMEM((ECH, HD), jnp.float32),
        pltpu.VMEM((ECH, 16), jnp.float32),
        pltpu.VMEM_SHARED((ACC_ROWS, HD), jnp.float32),
        pltpu.VMEM_SHARED((ACC_ROWS, 16), jnp.float32),
        pltpu.SemaphoreType.DMA,
    ],
    compiler_params=pltpu.CompilerParams(use_tc_tiling_on_sc=False),
)
def _sc_edges_cnt(h_hbm, src_hbm, dst_hbm, zrow_hbm, zc_hbm, ones_hbm,
                  acc_out, cnt_out, sidx_v, didx_v, rows_v, ones_v, acc_sh,
                  cnt_sh, gsem):
    _edge_body(True, h_hbm, src_hbm, dst_hbm, zrow_hbm, zc_hbm, ones_hbm,
               acc_out, cnt_out, sidx_v, didx_v, rows_v, ones_v, acc_sh,
               cnt_sh, gsem)


@functools.partial(
    pl.kernel,
    mesh=_mesh,
    out_type=jax.ShapeDtypeStruct((NC, ACC_ROWS, HD), jnp.float32),
    scratch_types=[
        pltpu.VMEM((ECH,), jnp.int32),
        pltpu.VMEM((ECH,), jnp.int32),
        pltpu.VMEM((ECH, HD), jnp.float32),
        pltpu.VMEM_SHARED((ACC_ROWS, HD), jnp.float32),
        pltpu.SemaphoreType.DMA,
    ],
    compiler_params=pltpu.CompilerParams(use_tc_tiling_on_sc=False),
)
def _sc_edges(h_hbm, src_hbm, dst_hbm, zrow_hbm, acc_out, sidx_v, didx_v,
              rows_v, acc_sh, gsem):
    _edge_body(False, h_hbm, src_hbm, dst_hbm, zrow_hbm, None, None,
               acc_out, None, sidx_v, didx_v, rows_v, None, acc_sh, None,
               gsem)


def _proj_body(x_ref, w_ref, b_ref, o_ref):
    o_ref[...] = jax.nn.relu(
        jnp.dot(x_ref[...], w_ref[...], preferred_element_type=jnp.float32)
        + b_ref[...])


def _proj(x, wT, b):
    return pl.pallas_call(
        _proj_body,
        grid=(10,),
        in_specs=[
            pl.BlockSpec((1000, D), lambda i: (i, 0)),
            pl.BlockSpec((D, D), lambda i: (0, 0)),
            pl.BlockSpec((1, D), lambda i: (0, 0)),
        ],
        out_specs=pl.BlockSpec((1000, D), lambda i: (i, 0)),
        out_shape=jax.ShapeDtypeStruct((N_NODES, D), jnp.float32),
    )(x, wT, b)


def _comb0_body(s_ref, c_ref, x_ref, wlT_ref, bl_ref, wrT_ref, g_ref, be_ref,
                o_ref):
    ssum = jnp.concatenate([s_ref[0], s_ref[1]], axis=-1)
    cnt = c_ref[0, :, 0:1]
    agg = ssum / jnp.maximum(cnt, 1.0)
    out = (jnp.dot(agg, wlT_ref[...], preferred_element_type=jnp.float32)
           + bl_ref[...]
           + jnp.dot(x_ref[...], wrT_ref[...],
                     preferred_element_type=jnp.float32))
    nrm = jnp.maximum(
        jnp.sqrt(jnp.sum(out * out, axis=-1, keepdims=True)), 1e-12)
    out = jax.nn.relu(out / nrm)
    m = jnp.mean(out, axis=-1, keepdims=True)
    v = jnp.mean((out - m) ** 2, axis=-1, keepdims=True)
    o_ref[...] = (out - m) / jnp.sqrt(v + 1e-5) * g_ref[...] + be_ref[...]


def _combine0(sparts, cparts, x, wlT, bl, wrT, g, be):
    return pl.pallas_call(
        _comb0_body,
        grid=(10,),
        in_specs=[
            pl.BlockSpec((NC, 1000, HD), lambda i: (0, i, 0)),
            pl.BlockSpec((1, 1000, 16), lambda i: (0, i, 0)),
            pl.BlockSpec((1000, D), lambda i: (i, 0)),
            pl.BlockSpec((D, D), lambda i: (0, 0)),
            pl.BlockSpec((1, D), lambda i: (0, 0)),
            pl.BlockSpec((D, D), lambda i: (0, 0)),
            pl.BlockSpec((1, D), lambda i: (0, 0)),
            pl.BlockSpec((1, D), lambda i: (0, 0)),
        ],
        out_specs=pl.BlockSpec((1000, D), lambda i: (i, 0)),
        out_shape=jax.ShapeDtypeStruct((N_NODES, D), jnp.float32),
    )(sparts, cparts, x, wlT, bl, wrT, g, be)


def _comb1_body(s_ref, c_ref, h_ref, wlT_ref, bl_ref, wrT_ref, o_ref):
    ssum = jnp.concatenate([s_ref[0], s_ref[1]], axis=-1)
    cnt = c_ref[0, :, 0:1]
    agg = ssum / jnp.maximum(cnt, 1.0)
    o_ref[...] = (jnp.dot(agg, wlT_ref[...], preferred_element_type=jnp.float32)
                  + bl_ref[...]
                  + jnp.dot(h_ref[...], wrT_ref[...],
                            preferred_element_type=jnp.float32))


def _combine1(sparts, cparts, h1, wlT, bl, wrT):
    return pl.pallas_call(
        _comb1_body,
        grid=(2,),
        in_specs=[
            pl.BlockSpec((NC, 1024, HD), lambda i: (0, i, 0)),
            pl.BlockSpec((1, 1024, 16), lambda i: (0, i, 0)),
            pl.BlockSpec((1024, D), lambda i: (i, 0)),
            pl.BlockSpec((D, D), lambda i: (0, 0)),
            pl.BlockSpec((1, D), lambda i: (0, 0)),
            pl.BlockSpec((D, D), lambda i: (0, 0)),
        ],
        out_specs=pl.BlockSpec((1024, D), lambda i: (i, 0)),
        out_shape=jax.ShapeDtypeStruct((BS, D), jnp.float32),
    )(sparts, cparts, h1, wlT, bl, wrT)


def _gru_body(lmax_ref, lens_ref, xpad, w0f, wh0f, b0f, bh0f, w0b, wh0b, b0b,
              bh0b, w1f, wh1f, b1f, bh1f, w1b, wh1b, b1b, bh1b,
              y0f, y0b, y1f, y1b, xt_v, af_v, ab_v, h_v, sem, wsem):
    lmax = lmax_ref[0]
    lens = lens_ref[...]

    def gru_pass(in_f, in_b, wi, wh, bi, bh, out_ref, reverse, mask_inputs):
        h_v[...] = jnp.zeros((NSEQ, HGRU), jnp.float32)

        def step(i, carry):
            t = (lmax - 1 - i) if reverse else i
            row = pl.multiple_of(t * NSEQ, NSEQ)
            if in_b is None:
                cpf = pltpu.make_async_copy(in_f.at[pl.ds(row, NSEQ), :],
                                            xt_v, sem)
                cpf.start()
                cpf.wait()
                xt = xt_v[...]
            else:
                cpf = pltpu.make_async_copy(in_f.at[pl.ds(row, NSEQ), :],
                                            af_v, sem)
                cpb = pltpu.make_async_copy(in_b.at[pl.ds(row, NSEQ), :],
                                            ab_v, sem)
                cpf.start()
                cpb.start()
                cpf.wait()
                cpb.wait()
                xt = jnp.concatenate([af_v[...], ab_v[...]], axis=1)
            if mask_inputs:
                xt = jnp.where(lens > t, xt, 0.0)
            h = h_v[...]
            gi = jnp.dot(xt, wi[...], preferred_element_type=jnp.float32) \
                + bi[...]
            gh = jnp.dot(h, wh[...], preferred_element_type=jnp.float32) \
                + bh[...]
            r = jax.nn.sigmoid(gi[:, :HGRU] + gh[:, :HGRU])
            z = jax.nn.sigmoid(gi[:, HGRU:2 * HGRU] + gh[:, HGRU:2 * HGRU])
            n = jnp.tanh(gi[:, 2 * HGRU:] + r * gh[:, 2 * HGRU:])
            h_v[...] = (1.0 - z) * n + z * h
            cp = pltpu.make_async_copy(h_v, out_ref.at[pl.ds(row, NSEQ), :],
                                       wsem)
            cp.start()
            cp.wait()
            return carry

        lax.fori_loop(0, lmax, step, 0)

    gru_pass(xpad, None, w0f, wh0f, b0f, bh0f, y0f, False, True)
    gru_pass(xpad, None, w0b, wh0b, b0b, bh0b, y0b, True, True)
    gru_pass(y0f, y0b, w1f, wh1f, b1f, bh1f, y1f, False, False)
    gru_pass(y0f, y0b, w1b, wh1b, b1b, bh1b, y1b, True, False)


def _gru(lmax, lens, xpad, weights):
    yshape = jax.ShapeDtypeStruct((PACK_ROWS, HGRU), jnp.float32)
    vspec = pl.BlockSpec(memory_space=pltpu.VMEM)
    return pl.pallas_call(
        _gru_body,
        in_specs=[pl.BlockSpec(memory_space=pltpu.SMEM),
                  vspec,
                  pl.BlockSpec(memory_space=pl.ANY)]
        + [vspec] * 16,
        out_specs=[pl.BlockSpec(memory_space=pl.ANY)] * 4,
        out_shape=[yshape] * 4,
        scratch_shapes=[
            pltpu.VMEM((NSEQ, D), jnp.float32),
            pltpu.VMEM((NSEQ, HGRU), jnp.float32),
            pltpu.VMEM((NSEQ, HGRU), jnp.float32),
            pltpu.VMEM((NSEQ, HGRU), jnp.float32),
            pltpu.SemaphoreType.DMA,
            pltpu.SemaphoreType.DMA,
        ],
    )(lmax, lens, xpad, *weights)


def _head_body(xf_ref, xb_ref, xg_ref, rg_ref, rb_ref, w1T_ref, b1_ref,
               mg_ref, mb_ref, w2T_ref, b2_ref, waT_ref, wbT_ref, bc_ref,
               o_ref):
    u = jnp.concatenate([xf_ref[...], xb_ref[...]], axis=1)
    m = jnp.mean(u, axis=-1, keepdims=True)
    v = jnp.mean((u - m) ** 2, axis=-1, keepdims=True)
    u = (u - m) / jnp.sqrt(v + 1e-5) * rg_ref[...] + rb_ref[...]
    u = jax.nn.relu(
        jnp.dot(u, w1T_ref[...], preferred_element_type=jnp.float32)
        + b1_ref[...])
    m = jnp.mean(u, axis=-1, keepdims=True)
    v = jnp.mean((u - m) ** 2, axis=-1, keepdims=True)
    u = (u - m) / jnp.sqrt(v + 1e-5) * mg_ref[...] + mb_ref[...]
    u = jnp.dot(u, w2T_ref[...], preferred_element_type=jnp.float32) \
        + b2_ref[...]
    o_ref[...] = (jnp.dot(u, waT_ref[...], preferred_element_type=jnp.float32)
                  + jnp.dot(xg_ref[...], wbT_ref[...],
                            preferred_element_type=jnp.float32)
                  + bc_ref[...])


def _head(xf, xb, xg, rg, rb, w1T, b1, mg, mb, w2T, b2, waT, wbT, bc):
    row = pl.BlockSpec((1024, D), lambda i: (i, 0))
    half = pl.BlockSpec((1024, HGRU), lambda i: (i, 0))
    wfull = pl.BlockSpec((D, D), lambda i: (0, 0))
    bfull = pl.BlockSpec((1, D), lambda i: (0, 0))
    return pl.pallas_call(
        _head_body,
        grid=(2,),
        in_specs=[half, half, row, bfull, bfull, wfull, bfull, bfull, bfull,
                  wfull, bfull, wfull, wfull, bfull],
        out_specs=row,
        out_shape=jax.ShapeDtypeStruct((BS, D), jnp.float32),
    )(xf, xb, xg, rg, rb, w1T, b1, mg, mb, w2T, b2, waT, wbT, bc)


def kernel(x, edge_index, batch, batch_size, neighbor_mask_node,
           neighbor_mask_edge, W_proj, b_proj, Wl0, bl0, Wr0, Wl1, bl1, Wr1,
           ln0_g, ln0_b, Wih0f, Whh0f, bih0f, bhh0f, Wih0b, Whh0b, bih0b,
           bhh0b, Wih1f, Whh1f, bih1f, bhh1f, Wih1b, Whh1b, bih1b, bhh1b,
           rnn_g, rnn_b, Wm1, bm1, mlp_g, mlp_b, Wm2, bm2, Wcat, bcat):
    f32 = jnp.float32
    x = x.astype(f32)
    bs = batch.shape[0]
    batch = batch.astype(jnp.int32)

    lengths = jnp.bincount(batch, length=NSEQ).astype(jnp.int32)
    starts = jnp.concatenate(
        [jnp.zeros((1,), jnp.int32), jnp.cumsum(lengths)[:-1].astype(jnp.int32)])
    pos = jnp.arange(bs, dtype=jnp.int32) - starts[batch]
    ridx = pos * NSEQ + batch
    lmax = jnp.max(lengths).reshape(1)
    sel = jnp.arange(bs) < batch_size
    x_t = jnp.where(sel[:, None], x[:bs], 0.0)

    xpad = _sc_pack(x_t, ridx)
    gw = (Wih0f.T, Whh0f.T, (bih0f).reshape(1, -1), (bhh0f).reshape(1, -1),
          Wih0b.T, Whh0b.T, (bih0b).reshape(1, -1), (bhh0b).reshape(1, -1),
          Wih1f.T, Whh1f.T, (bih1f).reshape(1, -1), (bhh1f).reshape(1, -1),
          Wih1b.T, Whh1b.T, (bih1b).reshape(1, -1), (bhh1b).reshape(1, -1))
    y0f, y0b, y1f, y1b = _gru(lmax.astype(jnp.int32),
                              lengths.reshape(NSEQ, 1), xpad,
                              [w.astype(f32) for w in gw])
    xrf, xrb = _sc_unpack(y1f, y1b, ridx)

    src = edge_index[0].astype(jnp.int32)
    dst = edge_index[1].astype(jnp.int32)
    epad = NS * EPS
    src_p = jnp.concatenate(
        [src, jnp.zeros((epad - src.shape[0],), jnp.int32)])
    dst_p = jnp.concatenate(
        [dst, jnp.full((epad - dst.shape[0],), N_NODES, jnp.int32)])
    zrow = jnp.zeros((ECH, HD), f32)
    zc = jnp.zeros((ECH, 16), f32)
    ones = jnp.ones((ECH, 16), f32)

    def halves(h):
        return jnp.concatenate([h[:, :HD], h[:, HD:]], axis=0)

    h0 = _proj(x, W_proj.T.astype(f32), b_proj.reshape(1, -1).astype(f32))
    s0, cnt = _sc_edges_cnt(halves(h0), src_p, dst_p, zrow, zc, ones)
    h1 = _combine0(s0, cnt, x, Wl0.T.astype(f32), bl0.reshape(1, -1),
                   Wr0.T.astype(f32), ln0_g.reshape(1, -1),
                   ln0_b.reshape(1, -1))
    s1 = _sc_edges(halves(h1), src_p, dst_p, zrow)
    x_gnn = _combine1(s1, cnt, h1[:BS], Wl1.T.astype(f32), bl1.reshape(1, -1),
                      Wr1.T.astype(f32))
    x_gnn = jnp.where(sel[:, None], x_gnn, 0.0)

    waT = Wcat[:, :D].T.astype(f32)
    wbT = Wcat[:, D:].T.astype(f32)
    return _head(xrf, xrb, x_gnn, rnn_g.reshape(1, -1), rnn_b.reshape(1, -1),
                 Wm1.T.astype(f32), bm1.reshape(1, -1), mlp_g.reshape(1, -1),
                 mlp_b.reshape(1, -1), Wm2.T.astype(f32), bm2.reshape(1, -1),
                 waT, wbT, bcat.reshape(1, -1))


# speedup vs baseline: 26.3508x; 1.4330x over previous
import functools

import jax
import jax.numpy as jnp
from jax import lax
from jax.experimental import pallas as pl
from jax.experimental.pallas import tpu as pltpu
from jax.experimental.pallas import tpu_sc as plsc

NSEQ = 64
BS = 2048
N_NODES = 10000
D = 128
HGRU = 64
E_EDGES = 320000

NC, NS = 2, 16
NW = NC * NS
HD = D // 2
EPS = 20480
ECH = 128
NCHUNK = EPS // ECH
ACC_ROWS = 10240
ROWS_PW = ACC_ROWS // NS
PACK_ROWS = BS * NSEQ

_mesh = plsc.VectorSubcoreMesh(core_axis_name="c", subcore_axis_name="s")


def _wid():
    return lax.axis_index("s") * NC + lax.axis_index("c")


@functools.partial(
    pl.kernel,
    mesh=_mesh,
    out_type=jax.ShapeDtypeStruct((PACK_ROWS, D), jnp.float32),
    scratch_types=[
        pltpu.VMEM((BS // NW,), jnp.int32),
        pltpu.VMEM((BS // NW, D), jnp.float32),
        pltpu.SemaphoreType.DMA,
    ],
)
def _sc_pack(x_hbm, ridx_hbm, out_hbm, idx_v, rows_v, sem):
    w = _wid()
    base = w * (BS // NW)
    pltpu.sync_copy(ridx_hbm.at[pl.ds(base, BS // NW)], idx_v)
    pltpu.sync_copy(x_hbm.at[pl.ds(base, BS // NW)], rows_v)
    pltpu.async_copy(rows_v, out_hbm.at[idx_v], sem).wait()


@functools.partial(
    pl.kernel,
    mesh=_mesh,
    out_type=(
        jax.ShapeDtypeStruct((BS, HGRU), jnp.float32),
        jax.ShapeDtypeStruct((BS, HGRU), jnp.float32),
    ),
    scratch_types=[
        pltpu.VMEM((BS // NW,), jnp.int32),
        pltpu.VMEM((BS // NW, HGRU), jnp.float32),
        pltpu.VMEM((BS // NW, HGRU), jnp.float32),
        pltpu.SemaphoreType.DMA,
        pltpu.SemaphoreType.DMA,
    ],
    compiler_params=pltpu.CompilerParams(use_tc_tiling_on_sc=False),
)
def _sc_unpack(yf_hbm, yb_hbm, gidx_hbm, of_hbm, ob_hbm, idx_v, gf_v, gb_v,
               semf, semb):
    w = _wid()
    k = BS // NW
    base = w * k
    pltpu.sync_copy(gidx_hbm.at[pl.ds(base, k)], idx_v)
    cf = pltpu.async_copy(yf_hbm.at[idx_v], gf_v, semf)
    cb = pltpu.async_copy(yb_hbm.at[idx_v], gb_v, semb)
    cf.wait()
    cb.wait()
    pltpu.sync_copy(gf_v, of_hbm.at[pl.ds(base, k)])
    pltpu.sync_copy(gb_v, ob_hbm.at[pl.ds(base, k)])


def _edge_body(with_cnt, h_hbm, src_hbm, dst_hbm, zrow_hbm, zc_hbm, ones_hbm,
               acc_out, cnt_out, sidx_v, didx_v, rows_v, ones_v, acc_sh,
               cnt_sh, gsem0, gsem1):
    c = lax.axis_index("c")
    s = lax.axis_index("s")

    pltpu.sync_copy(zrow_hbm, rows_v.at[0])
    if with_cnt:
        pltpu.sync_copy(zc_hbm, ones_v)
    for j in range(ROWS_PW // ECH):
        pltpu.sync_copy(rows_v.at[0],
                        acc_sh.at[pl.ds(s * ROWS_PW + j * ECH, ECH)])
        if with_cnt:
            pltpu.sync_copy(ones_v,
                            cnt_sh.at[pl.ds(s * ROWS_PW + j * ECH, ECH)])
    if with_cnt:
        pltpu.sync_copy(ones_hbm, ones_v)
    pltpu.sync_copy(src_hbm.at[c, s], sidx_v)
    pltpu.sync_copy(dst_hbm.at[s], didx_v)
    plsc.subcore_barrier()

    sems = (gsem0, gsem1)
    pltpu.async_copy(h_hbm.at[sidx_v.at[0]], rows_v.at[0], gsem0)
    pltpu.async_copy(h_hbm.at[sidx_v.at[1]], rows_v.at[1], gsem1)

    def pair(i, carry):
        for b in range(2):
            j = 2 * i + b
            pltpu.make_async_copy(h_hbm.at[sidx_v.at[j]], rows_v.at[b],
                                  sems[b]).wait()
            pltpu.sync_copy(rows_v.at[b], acc_sh.at[didx_v.at[j]], add=True)
            if with_cnt:
                pltpu.sync_copy(ones_v, cnt_sh.at[didx_v.at[j]], add=True)

            @pl.when(j + 2 < NCHUNK)
            def _():
                pltpu.async_copy(h_hbm.at[sidx_v.at[j + 2]], rows_v.at[b],
                                 sems[b])
        return carry

    lax.fori_loop(0, NCHUNK // 2, pair, 0)
    plsc.subcore_barrier()

    rbase = s * ROWS_PW
    pltpu.sync_copy(acc_sh.at[pl.ds(rbase, ROWS_PW)],
                    acc_out.at[c, pl.ds(rbase, ROWS_PW)])
    if with_cnt:
        pltpu.sync_copy(cnt_sh.at[pl.ds(rbase, ROWS_PW)],
                        cnt_out.at[c, pl.ds(rbase, ROWS_PW)])


@functools.partial(
    pl.kernel,
    mesh=_mesh,
    out_type=(
        jax.ShapeDtypeStruct((NC, ACC_ROWS, HD), jnp.float32),
        jax.ShapeDtypeStruct((NC, ACC_ROWS, 16), jnp.float32),
    ),
    scratch_types=[
        pltpu.VMEM((NCHUNK, ECH), jnp.int32),
        pltpu.VMEM((NCHUNK, ECH), jnp.int32),
        pltpu.VMEM((2, ECH, HD), jnp.float32),
        pltpu.VMEM((ECH, 16), jnp.float32),
        pltpu.VMEM_SHARED((ACC_ROWS, HD), jnp.float32),
        pltpu.VMEM_SHARED((ACC_ROWS, 16), jnp.float32),
        pltpu.SemaphoreType.DMA,
        pltpu.SemaphoreType.DMA,
    ],
    compiler_params=pltpu.CompilerParams(use_tc_tiling_on_sc=False),
)
def _sc_edges_cnt(h_hbm, src_hbm, dst_hbm, zrow_hbm, zc_hbm, ones_hbm,
                  acc_out, cnt_out, sidx_v, didx_v, rows_v, ones_v, acc_sh,
                  cnt_sh, gsem0, gsem1):
    _edge_body(True, h_hbm, src_hbm, dst_hbm, zrow_hbm, zc_hbm, ones_hbm,
               acc_out, cnt_out, sidx_v, didx_v, rows_v, ones_v, acc_sh,
               cnt_sh, gsem0, gsem1)


@functools.partial(
    pl.kernel,
    mesh=_mesh,
    out_type=jax.ShapeDtypeStruct((NC, ACC_ROWS, HD), jnp.float32),
    scratch_types=[
        pltpu.VMEM((NCHUNK, ECH), jnp.int32),
        pltpu.VMEM((NCHUNK, ECH), jnp.int32),
        pltpu.VMEM((2, ECH, HD), jnp.float32),
        pltpu.VMEM_SHARED((ACC_ROWS, HD), jnp.float32),
        pltpu.SemaphoreType.DMA,
        pltpu.SemaphoreType.DMA,
    ],
    compiler_params=pltpu.CompilerParams(use_tc_tiling_on_sc=False),
)
def _sc_edges(h_hbm, src_hbm, dst_hbm, zrow_hbm, acc_out, sidx_v, didx_v,
              rows_v, acc_sh, gsem0, gsem1):
    _edge_body(False, h_hbm, src_hbm, dst_hbm, zrow_hbm, None, None,
               acc_out, None, sidx_v, didx_v, rows_v, None, acc_sh, None,
               gsem0, gsem1)


def _proj_body(x_ref, w_ref, b_ref, o_ref):
    o_ref[...] = jax.nn.relu(
        jnp.dot(x_ref[...], w_ref[...], preferred_element_type=jnp.float32)
        + b_ref[...])


def _proj(x, wT, b):
    return pl.pallas_call(
        _proj_body,
        grid=(10,),
        in_specs=[
            pl.BlockSpec((1000, D), lambda i: (i, 0)),
            pl.BlockSpec((D, D), lambda i: (0, 0)),
            pl.BlockSpec((1, D), lambda i: (0, 0)),
        ],
        out_specs=pl.BlockSpec((1000, D), lambda i: (i, 0)),
        out_shape=jax.ShapeDtypeStruct((N_NODES, D), jnp.float32),
    )(x, wT, b)


def _comb0_body(s_ref, c_ref, x_ref, wlT_ref, bl_ref, wrT_ref, g_ref, be_ref,
                o_ref):
    ssum = jnp.concatenate([s_ref[0], s_ref[1]], axis=-1)
    cnt = c_ref[0, :, 0:1]
    agg = ssum / jnp.maximum(cnt, 1.0)
    out = (jnp.dot(agg, wlT_ref[...], preferred_element_type=jnp.float32)
           + bl_ref[...]
           + jnp.dot(x_ref[...], wrT_ref[...],
                     preferred_element_type=jnp.float32))
    nrm = jnp.maximum(
        jnp.sqrt(jnp.sum(out * out, axis=-1, keepdims=True)), 1e-12)
    out = jax.nn.relu(out / nrm)
    m = jnp.mean(out, axis=-1, keepdims=True)
    v = jnp.mean((out - m) ** 2, axis=-1, keepdims=True)
    o_ref[...] = (out - m) / jnp.sqrt(v + 1e-5) * g_ref[...] + be_ref[...]


def _combine0(sparts, cparts, x, wlT, bl, wrT, g, be):
    return pl.pallas_call(
        _comb0_body,
        grid=(10,),
        in_specs=[
            pl.BlockSpec((NC, 1000, HD), lambda i: (0, i, 0)),
            pl.BlockSpec((1, 1000, 16), lambda i: (0, i, 0)),
            pl.BlockSpec((1000, D), lambda i: (i, 0)),
            pl.BlockSpec((D, D), lambda i: (0, 0)),
            pl.BlockSpec((1, D), lambda i: (0, 0)),
            pl.BlockSpec((D, D), lambda i: (0, 0)),
            pl.BlockSpec((1, D), lambda i: (0, 0)),
            pl.BlockSpec((1, D), lambda i: (0, 0)),
        ],
        out_specs=pl.BlockSpec((1000, D), lambda i: (i, 0)),
        out_shape=jax.ShapeDtypeStruct((N_NODES, D), jnp.float32),
    )(sparts, cparts, x, wlT, bl, wrT, g, be)


def _comb1_body(s_ref, c_ref, h_ref, wlT_ref, bl_ref, wrT_ref, o_ref):
    ssum = jnp.concatenate([s_ref[0], s_ref[1]], axis=-1)
    cnt = c_ref[0, :, 0:1]
    agg = ssum / jnp.maximum(cnt, 1.0)
    o_ref[...] = (jnp.dot(agg, wlT_ref[...], preferred_element_type=jnp.float32)
                  + bl_ref[...]
                  + jnp.dot(h_ref[...], wrT_ref[...],
                            preferred_element_type=jnp.float32))


def _combine1(sparts, cparts, h1, wlT, bl, wrT):
    return pl.pallas_call(
        _comb1_body,
        grid=(2,),
        in_specs=[
            pl.BlockSpec((NC, 1024, HD), lambda i: (0, i, 0)),
            pl.BlockSpec((1, 1024, 16), lambda i: (0, i, 0)),
            pl.BlockSpec((1024, D), lambda i: (i, 0)),
            pl.BlockSpec((D, D), lambda i: (0, 0)),
            pl.BlockSpec((1, D), lambda i: (0, 0)),
            pl.BlockSpec((D, D), lambda i: (0, 0)),
        ],
        out_specs=pl.BlockSpec((1024, D), lambda i: (i, 0)),
        out_shape=jax.ShapeDtypeStruct((BS, D), jnp.float32),
    )(sparts, cparts, h1, wlT, bl, wrT)


def _gru_body(lmax_ref, lens_ref, xpad, w0f, wh0f, b0f, bh0f, w0b, wh0b, b0b,
              bh0b, w1f, wh1f, b1f, bh1f, w1b, wh1b, b1b, bh1b,
              y0f, y0b, y1f, y1b, xt_v, af_v, ab_v, h_v, sem, wsem):
    lmax = lmax_ref[0]
    lens = lens_ref[...]

    def gru_pass(in_f, in_b, wi, wh, bi, bh, out_ref, reverse, mask_inputs):
        h_v[...] = jnp.zeros((NSEQ, HGRU), jnp.float32)

        def step(i, carry):
            t = (lmax - 1 - i) if reverse else i
            row = pl.multiple_of(t * NSEQ, NSEQ)
            if in_b is None:
                cpf = pltpu.make_async_copy(in_f.at[pl.ds(row, NSEQ), :],
                                            xt_v, sem)
                cpf.start()
                cpf.wait()
                xt = xt_v[...]
            else:
                cpf = pltpu.make_async_copy(in_f.at[pl.ds(row, NSEQ), :],
                                            af_v, sem)
                cpb = pltpu.make_async_copy(in_b.at[pl.ds(row, NSEQ), :],
                                            ab_v, sem)
                cpf.start()
                cpb.start()
                cpf.wait()
                cpb.wait()
                xt = jnp.concatenate([af_v[...], ab_v[...]], axis=1)
            if mask_inputs:
                xt = jnp.where(lens > t, xt, 0.0)
            h = h_v[...]
            gi = jnp.dot(xt, wi[...], preferred_element_type=jnp.float32) \
                + bi[...]
            gh = jnp.dot(h, wh[...], preferred_element_type=jnp.float32) \
                + bh[...]
            r = jax.nn.sigmoid(gi[:, :HGRU] + gh[:, :HGRU])
            z = jax.nn.sigmoid(gi[:, HGRU:2 * HGRU] + gh[:, HGRU:2 * HGRU])
            n = jnp.tanh(gi[:, 2 * HGRU:] + r * gh[:, 2 * HGRU:])
            h_v[...] = (1.0 - z) * n + z * h
            cp = pltpu.make_async_copy(h_v, out_ref.at[pl.ds(row, NSEQ), :],
                                       wsem)
            cp.start()
            cp.wait()
            return carry

        lax.fori_loop(0, lmax, step, 0)

    gru_pass(xpad, None, w0f, wh0f, b0f, bh0f, y0f, False, True)
    gru_pass(xpad, None, w0b, wh0b, b0b, bh0b, y0b, True, True)
    gru_pass(y0f, y0b, w1f, wh1f, b1f, bh1f, y1f, False, False)
    gru_pass(y0f, y0b, w1b, wh1b, b1b, bh1b, y1b, True, False)


def _gru(lmax, lens, xpad, weights):
    yshape = jax.ShapeDtypeStruct((PACK_ROWS, HGRU), jnp.float32)
    vspec = pl.BlockSpec(memory_space=pltpu.VMEM)
    return pl.pallas_call(
        _gru_body,
        in_specs=[pl.BlockSpec(memory_space=pltpu.SMEM),
                  vspec,
                  pl.BlockSpec(memory_space=pl.ANY)]
        + [vspec] * 16,
        out_specs=[pl.BlockSpec(memory_space=pl.ANY)] * 4,
        out_shape=[yshape] * 4,
        scratch_shapes=[
            pltpu.VMEM((NSEQ, D), jnp.float32),
            pltpu.VMEM((NSEQ, HGRU), jnp.float32),
            pltpu.VMEM((NSEQ, HGRU), jnp.float32),
            pltpu.VMEM((NSEQ, HGRU), jnp.float32),
            pltpu.SemaphoreType.DMA,
            pltpu.SemaphoreType.DMA,
        ],
    )(lmax, lens, xpad, *weights)


def _head_body(xf_ref, xb_ref, xg_ref, rg_ref, rb_ref, w1T_ref, b1_ref,
               mg_ref, mb_ref, w2T_ref, b2_ref, waT_ref, wbT_ref, bc_ref,
               o_ref):
    u = jnp.concatenate([xf_ref[...], xb_ref[...]], axis=1)
    m = jnp.mean(u, axis=-1, keepdims=True)
    v = jnp.mean((u - m) ** 2, axis=-1, keepdims=True)
    u = (u - m) / jnp.sqrt(v + 1e-5) * rg_ref[...] + rb_ref[...]
    u = jax.nn.relu(
        jnp.dot(u, w1T_ref[...], preferred_element_type=jnp.float32)
        + b1_ref[...])
    m = jnp.mean(u, axis=-1, keepdims=True)
    v = jnp.mean((u - m) ** 2, axis=-1, keepdims=True)
    u = (u - m) / jnp.sqrt(v + 1e-5) * mg_ref[...] + mb_ref[...]
    u = jnp.dot(u, w2T_ref[...], preferred_element_type=jnp.float32) \
        + b2_ref[...]
    o_ref[...] = (jnp.dot(u, waT_ref[...], preferred_element_type=jnp.float32)
                  + jnp.dot(xg_ref[...], wbT_ref[...],
                            preferred_element_type=jnp.float32)
                  + bc_ref[...])


def _head(xf, xb, xg, rg, rb, w1T, b1, mg, mb, w2T, b2, waT, wbT, bc):
    row = pl.BlockSpec((1024, D), lambda i: (i, 0))
    half = pl.BlockSpec((1024, HGRU), lambda i: (i, 0))
    wfull = pl.BlockSpec((D, D), lambda i: (0, 0))
    bfull = pl.BlockSpec((1, D), lambda i: (0, 0))
    return pl.pallas_call(
        _head_body,
        grid=(2,),
        in_specs=[half, half, row, bfull, bfull, wfull, bfull, bfull, bfull,
                  wfull, bfull, wfull, wfull, bfull],
        out_specs=row,
        out_shape=jax.ShapeDtypeStruct((BS, D), jnp.float32),
    )(xf, xb, xg, rg, rb, w1T, b1, mg, mb, w2T, b2, waT, wbT, bc)


def kernel(x, edge_index, batch, batch_size, neighbor_mask_node,
           neighbor_mask_edge, W_proj, b_proj, Wl0, bl0, Wr0, Wl1, bl1, Wr1,
           ln0_g, ln0_b, Wih0f, Whh0f, bih0f, bhh0f, Wih0b, Whh0b, bih0b,
           bhh0b, Wih1f, Whh1f, bih1f, bhh1f, Wih1b, Whh1b, bih1b, bhh1b,
           rnn_g, rnn_b, Wm1, bm1, mlp_g, mlp_b, Wm2, bm2, Wcat, bcat):
    f32 = jnp.float32
    x = x.astype(f32)
    bs = batch.shape[0]
    batch = batch.astype(jnp.int32)

    lengths = jnp.bincount(batch, length=NSEQ).astype(jnp.int32)
    starts = jnp.concatenate(
        [jnp.zeros((1,), jnp.int32), jnp.cumsum(lengths)[:-1].astype(jnp.int32)])
    pos = jnp.arange(bs, dtype=jnp.int32) - starts[batch]
    ridx = pos * NSEQ + batch
    lmax = jnp.max(lengths).reshape(1)
    sel = jnp.arange(bs) < batch_size
    x_t = jnp.where(sel[:, None], x[:bs], 0.0)

    xpad = _sc_pack(x_t, ridx)
    gw = (Wih0f.T, Whh0f.T, (bih0f).reshape(1, -1), (bhh0f).reshape(1, -1),
          Wih0b.T, Whh0b.T, (bih0b).reshape(1, -1), (bhh0b).reshape(1, -1),
          Wih1f.T, Whh1f.T, (bih1f).reshape(1, -1), (bhh1f).reshape(1, -1),
          Wih1b.T, Whh1b.T, (bih1b).reshape(1, -1), (bhh1b).reshape(1, -1))
    y0f, y0b, y1f, y1b = _gru(lmax.astype(jnp.int32),
                              lengths.reshape(NSEQ, 1), xpad,
                              [w.astype(f32) for w in gw])
    xrf, xrb = _sc_unpack(y1f, y1b, ridx)

    src = edge_index[0].astype(jnp.int32)
    dst = edge_index[1].astype(jnp.int32)
    epad = NS * EPS
    src_p = jnp.concatenate(
        [src, jnp.zeros((epad - src.shape[0],), jnp.int32)])
    dst_p = jnp.concatenate(
        [dst, jnp.full((epad - dst.shape[0],), N_NODES, jnp.int32)])
    src_r = src_p.reshape(NS, NCHUNK, ECH)
    src_p = jnp.stack([src_r, src_r + N_NODES])
    dst_p = dst_p.reshape(NS, NCHUNK, ECH)
    zrow = jnp.zeros((ECH, HD), f32)
    zc = jnp.zeros((ECH, 16), f32)
    ones = jnp.ones((ECH, 16), f32)

    def halves(h):
        return jnp.concatenate([h[:, :HD], h[:, HD:]], axis=0)

    h0 = _proj(x, W_proj.T.astype(f32), b_proj.reshape(1, -1).astype(f32))
    s0, cnt = _sc_edges_cnt(halves(h0), src_p, dst_p, zrow, zc, ones)
    h1 = _combine0(s0, cnt, x, Wl0.T.astype(f32), bl0.reshape(1, -1),
                   Wr0.T.astype(f32), ln0_g.reshape(1, -1),
                   ln0_b.reshape(1, -1))
    s1 = _sc_edges(halves(h1), src_p, dst_p, zrow)
    x_gnn = _combine1(s1, cnt, h1[:BS], Wl1.T.astype(f32), bl1.reshape(1, -1),
                      Wr1.T.astype(f32))
    x_gnn = jnp.where(sel[:, None], x_gnn, 0.0)

    waT = Wcat[:, :D].T.astype(f32)
    wbT = Wcat[:, D:].T.astype(f32)
    return _head(xrf, xrb, x_gnn, rnn_g.reshape(1, -1), rnn_b.reshape(1, -1),
                 Wm1.T.astype(f32), bm1.reshape(1, -1), mlp_g.reshape(1, -1),
                 mlp_b.reshape(1, -1), Wm2.T.astype(f32), bm2.reshape(1, -1),
                 waT, wbT, bcat.reshape(1, -1))


# speedup vs baseline: 33.9357x; 1.2878x over previous
import functools

import jax
import jax.numpy as jnp
from jax import lax
from jax.experimental import pallas as pl
from jax.experimental.pallas import tpu as pltpu
from jax.experimental.pallas import tpu_sc as plsc

NSEQ = 64
BS = 2048
N_NODES = 10000
D = 128
HGRU = 64
E_EDGES = 320000

NC, NS = 2, 16
NW = NC * NS
HD = D // 2
EPS = 20480
ECH = 128
NCHUNK = EPS // ECH
ACC_ROWS = 10240
ROWS_PW = ACC_ROWS // NS
PACK_ROWS = BS * NSEQ

_mesh = plsc.VectorSubcoreMesh(core_axis_name="c", subcore_axis_name="s")


def _wid():
    return lax.axis_index("s") * NC + lax.axis_index("c")


@functools.partial(
    pl.kernel,
    mesh=_mesh,
    out_type=jax.ShapeDtypeStruct((PACK_ROWS, D), jnp.float32),
    scratch_types=[
        pltpu.VMEM((BS // NW,), jnp.int32),
        pltpu.VMEM((BS // NW, D), jnp.float32),
        pltpu.SemaphoreType.DMA,
    ],
)
def _sc_pack(x_hbm, ridx_hbm, out_hbm, idx_v, rows_v, sem):
    w = _wid()
    base = w * (BS // NW)
    pltpu.sync_copy(ridx_hbm.at[pl.ds(base, BS // NW)], idx_v)
    pltpu.sync_copy(x_hbm.at[pl.ds(base, BS // NW)], rows_v)
    pltpu.async_copy(rows_v, out_hbm.at[idx_v], sem).wait()


@functools.partial(
    pl.kernel,
    mesh=_mesh,
    out_type=(
        jax.ShapeDtypeStruct((BS, HGRU), jnp.float32),
        jax.ShapeDtypeStruct((BS, HGRU), jnp.float32),
    ),
    scratch_types=[
        pltpu.VMEM((BS // NW,), jnp.int32),
        pltpu.VMEM((BS // NW, HGRU), jnp.float32),
        pltpu.VMEM((BS // NW, HGRU), jnp.float32),
        pltpu.SemaphoreType.DMA,
        pltpu.SemaphoreType.DMA,
    ],
    compiler_params=pltpu.CompilerParams(use_tc_tiling_on_sc=False),
)
def _sc_unpack(yf_hbm, yb_hbm, gidx_hbm, of_hbm, ob_hbm, idx_v, gf_v, gb_v,
               semf, semb):
    w = _wid()
    k = BS // NW
    base = w * k
    pltpu.sync_copy(gidx_hbm.at[pl.ds(base, k)], idx_v)
    cf = pltpu.async_copy(yf_hbm.at[idx_v], gf_v, semf)
    cb = pltpu.async_copy(yb_hbm.at[idx_v], gb_v, semb)
    cf.wait()
    cb.wait()
    pltpu.sync_copy(gf_v, of_hbm.at[pl.ds(base, k)])
    pltpu.sync_copy(gb_v, ob_hbm.at[pl.ds(base, k)])


def _edge_body(with_cnt, h_hbm, src_hbm, dst_hbm, zrow_hbm, zc_hbm, ones_hbm,
               acc_out, cnt_out, sidx_v, didx_v, rows_v, ones_v, acc_sh,
               cnt_sh, gsem0, gsem1):
    c = lax.axis_index("c")
    s = lax.axis_index("s")

    pltpu.sync_copy(zrow_hbm, rows_v.at[0])
    if with_cnt:
        pltpu.sync_copy(zc_hbm, ones_v)
    for j in range(ROWS_PW // ECH):
        pltpu.sync_copy(rows_v.at[0],
                        acc_sh.at[pl.ds(s * ROWS_PW + j * ECH, ECH)])
        if with_cnt:
            pltpu.sync_copy(ones_v,
                            cnt_sh.at[pl.ds(s * ROWS_PW + j * ECH, ECH)])
    if with_cnt:
        pltpu.sync_copy(ones_hbm, ones_v)
    pltpu.sync_copy(src_hbm.at[c, s], sidx_v)
    pltpu.sync_copy(dst_hbm.at[s], didx_v)
    plsc.subcore_barrier()

    sems = (gsem0, gsem1)
    pltpu.async_copy(h_hbm.at[sidx_v.at[0]], rows_v.at[0], gsem0)
    pltpu.async_copy(h_hbm.at[sidx_v.at[1]], rows_v.at[1], gsem1)

    def pair(i, carry):
        for b in range(2):
            j = 2 * i + b
            pltpu.make_async_copy(h_hbm.at[sidx_v.at[j]], rows_v.at[b],
                                  sems[b]).wait()
            pltpu.sync_copy(rows_v.at[b], acc_sh.at[didx_v.at[j]], add=True)
            if with_cnt:
                pltpu.sync_copy(ones_v, cnt_sh.at[didx_v.at[j]], add=True)

            @pl.when(j + 2 < NCHUNK)
            def _():
                pltpu.async_copy(h_hbm.at[sidx_v.at[j + 2]], rows_v.at[b],
                                 sems[b])
        return carry

    lax.fori_loop(0, NCHUNK // 2, pair, 0)
    plsc.subcore_barrier()

    rbase = s * ROWS_PW
    pltpu.sync_copy(acc_sh.at[pl.ds(rbase, ROWS_PW)],
                    acc_out.at[c, pl.ds(rbase, ROWS_PW)])
    if with_cnt:
        pltpu.sync_copy(cnt_sh.at[pl.ds(rbase, ROWS_PW)],
                        cnt_out.at[c, pl.ds(rbase, ROWS_PW)])


@functools.partial(
    pl.kernel,
    mesh=_mesh,
    out_type=(
        jax.ShapeDtypeStruct((NC, ACC_ROWS, HD), jnp.float32),
        jax.ShapeDtypeStruct((NC, ACC_ROWS, 16), jnp.float32),
    ),
    scratch_types=[
        pltpu.VMEM((NCHUNK, ECH), jnp.int32),
        pltpu.VMEM((NCHUNK, ECH), jnp.int32),
        pltpu.VMEM((2, ECH, HD), jnp.float32),
        pltpu.VMEM((ECH, 16), jnp.float32),
        pltpu.VMEM_SHARED((ACC_ROWS, HD), jnp.float32),
        pltpu.VMEM_SHARED((ACC_ROWS, 16), jnp.float32),
        pltpu.SemaphoreType.DMA,
        pltpu.SemaphoreType.DMA,
    ],
    compiler_params=pltpu.CompilerParams(use_tc_tiling_on_sc=False),
)
def _sc_edges_cnt(h_hbm, src_hbm, dst_hbm, zrow_hbm, zc_hbm, ones_hbm,
                  acc_out, cnt_out, sidx_v, didx_v, rows_v, ones_v, acc_sh,
                  cnt_sh, gsem0, gsem1):
    _edge_body(True, h_hbm, src_hbm, dst_hbm, zrow_hbm, zc_hbm, ones_hbm,
               acc_out, cnt_out, sidx_v, didx_v, rows_v, ones_v, acc_sh,
               cnt_sh, gsem0, gsem1)


@functools.partial(
    pl.kernel,
    mesh=_mesh,
    out_type=jax.ShapeDtypeStruct((NC, ACC_ROWS, HD), jnp.float32),
    scratch_types=[
        pltpu.VMEM((NCHUNK, ECH), jnp.int32),
        pltpu.VMEM((NCHUNK, ECH), jnp.int32),
        pltpu.VMEM((2, ECH, HD), jnp.float32),
        pltpu.VMEM_SHARED((ACC_ROWS, HD), jnp.float32),
        pltpu.SemaphoreType.DMA,
        pltpu.SemaphoreType.DMA,
    ],
    compiler_params=pltpu.CompilerParams(use_tc_tiling_on_sc=False),
)
def _sc_edges(h_hbm, src_hbm, dst_hbm, zrow_hbm, acc_out, sidx_v, didx_v,
              rows_v, acc_sh, gsem0, gsem1):
    _edge_body(False, h_hbm, src_hbm, dst_hbm, zrow_hbm, None, None,
               acc_out, None, sidx_v, didx_v, rows_v, None, acc_sh, None,
               gsem0, gsem1)


def _proj_body(x_ref, w_ref, b_ref, o_ref):
    o_ref[...] = jax.nn.relu(
        jnp.dot(x_ref[...], w_ref[...], preferred_element_type=jnp.float32)
        + b_ref[...])


def _proj(x, wT, b):
    return pl.pallas_call(
        _proj_body,
        grid=(10,),
        in_specs=[
            pl.BlockSpec((1000, D), lambda i: (i, 0)),
            pl.BlockSpec((D, D), lambda i: (0, 0)),
            pl.BlockSpec((1, D), lambda i: (0, 0)),
        ],
        out_specs=pl.BlockSpec((1000, D), lambda i: (i, 0)),
        out_shape=jax.ShapeDtypeStruct((N_NODES, D), jnp.float32),
    )(x, wT, b)


def _comb0_body(s_ref, c_ref, x_ref, wlT_ref, bl_ref, wrT_ref, g_ref, be_ref,
                o_ref):
    ssum = jnp.concatenate([s_ref[0], s_ref[1]], axis=-1)
    cnt = c_ref[0, :, 0:1]
    agg = ssum / jnp.maximum(cnt, 1.0)
    out = (jnp.dot(agg, wlT_ref[...], preferred_element_type=jnp.float32)
           + bl_ref[...]
           + jnp.dot(x_ref[...], wrT_ref[...],
                     preferred_element_type=jnp.float32))
    nrm = jnp.maximum(
        jnp.sqrt(jnp.sum(out * out, axis=-1, keepdims=True)), 1e-12)
    out = jax.nn.relu(out / nrm)
    m = jnp.mean(out, axis=-1, keepdims=True)
    v = jnp.mean((out - m) ** 2, axis=-1, keepdims=True)
    o_ref[...] = (out - m) / jnp.sqrt(v + 1e-5) * g_ref[...] + be_ref[...]


def _combine0(sparts, cparts, x, wlT, bl, wrT, g, be):
    return pl.pallas_call(
        _comb0_body,
        grid=(10,),
        in_specs=[
            pl.BlockSpec((NC, 1000, HD), lambda i: (0, i, 0)),
            pl.BlockSpec((1, 1000, 16), lambda i: (0, i, 0)),
            pl.BlockSpec((1000, D), lambda i: (i, 0)),
            pl.BlockSpec((D, D), lambda i: (0, 0)),
            pl.BlockSpec((1, D), lambda i: (0, 0)),
            pl.BlockSpec((D, D), lambda i: (0, 0)),
            pl.BlockSpec((1, D), lambda i: (0, 0)),
            pl.BlockSpec((1, D), lambda i: (0, 0)),
        ],
        out_specs=pl.BlockSpec((1000, D), lambda i: (i, 0)),
        out_shape=jax.ShapeDtypeStruct((N_NODES, D), jnp.float32),
    )(sparts, cparts, x, wlT, bl, wrT, g, be)


def _comb1_body(s_ref, c_ref, h_ref, wlT_ref, bl_ref, wrT_ref, o_ref):
    ssum = jnp.concatenate([s_ref[0], s_ref[1]], axis=-1)
    cnt = c_ref[0, :, 0:1]
    agg = ssum / jnp.maximum(cnt, 1.0)
    o_ref[...] = (jnp.dot(agg, wlT_ref[...], preferred_element_type=jnp.float32)
                  + bl_ref[...]
                  + jnp.dot(h_ref[...], wrT_ref[...],
                            preferred_element_type=jnp.float32))


def _combine1(sparts, cparts, h1, wlT, bl, wrT):
    return pl.pallas_call(
        _comb1_body,
        grid=(2,),
        in_specs=[
            pl.BlockSpec((NC, 1024, HD), lambda i: (0, i, 0)),
            pl.BlockSpec((1, 1024, 16), lambda i: (0, i, 0)),
            pl.BlockSpec((1024, D), lambda i: (i, 0)),
            pl.BlockSpec((D, D), lambda i: (0, 0)),
            pl.BlockSpec((1, D), lambda i: (0, 0)),
            pl.BlockSpec((D, D), lambda i: (0, 0)),
        ],
        out_specs=pl.BlockSpec((1024, D), lambda i: (i, 0)),
        out_shape=jax.ShapeDtypeStruct((BS, D), jnp.float32),
    )(sparts, cparts, h1, wlT, bl, wrT)


CT = 64


def _gru_body(lmax_ref, lens_ref, xpad, w0f, wh0f, b0f, bh0f, w0b, wh0b, b0b,
              bh0b, w1f, wh1f, b1f, bh1f, w1b, wh1b, b1b, bh1b,
              y0f, y0b, y1f, y1b, xch_v, af_v, ab_v, ych_v, h_v, sem, wsem):
    lmax = lmax_ref[0]
    lens = lens_ref[...]
    nch = (lmax + CT - 1) // CT

    def gru_pass(in_f, in_b, wi, wh, bi, bh, out_ref, reverse, mask_inputs):
        h_v[...] = jnp.zeros((NSEQ, HGRU), jnp.float32)

        def chunk(ci, carry):
            cc = (nch - 1 - ci) if reverse else ci
            base_t = cc * CT
            rem = jnp.minimum(lmax - base_t, CT)
            rows0 = pl.multiple_of(base_t * NSEQ, NSEQ)
            if in_b is None:
                cpf = pltpu.make_async_copy(
                    in_f.at[pl.ds(rows0, CT * NSEQ), :], xch_v, sem)
                cpf.start()
                cpf.wait()
            else:
                cpf = pltpu.make_async_copy(
                    in_f.at[pl.ds(rows0, CT * NSEQ), :], af_v, sem)
                cpb = pltpu.make_async_copy(
                    in_b.at[pl.ds(rows0, CT * NSEQ), :], ab_v, sem)
                cpf.start()
                cpb.start()
                cpf.wait()
                cpb.wait()

            def step(tt, c2):
                si = (rem - 1 - tt) if reverse else tt
                r0 = si * NSEQ
                if in_b is None:
                    xt = xch_v[pl.ds(r0, NSEQ), :]
                else:
                    xt = jnp.concatenate(
                        [af_v[pl.ds(r0, NSEQ), :], ab_v[pl.ds(r0, NSEQ), :]],
                        axis=1)
                if mask_inputs:
                    xt = jnp.where(lens > base_t + si, xt, 0.0)
                h = h_v[...]
                gi = jnp.dot(xt, wi[...],
                             preferred_element_type=jnp.float32) + bi[...]
                gh = jnp.dot(h, wh[...],
                             preferred_element_type=jnp.float32) + bh[...]
                r = jax.nn.sigmoid(gi[:, :HGRU] + gh[:, :HGRU])
                z = jax.nn.sigmoid(gi[:, HGRU:2 * HGRU]
                                   + gh[:, HGRU:2 * HGRU])
                n = jnp.tanh(gi[:, 2 * HGRU:] + r * gh[:, 2 * HGRU:])
                hn = (1.0 - z) * n + z * h
                h_v[...] = hn
                ych_v[pl.ds(r0, NSEQ), :] = hn
                return c2

            lax.fori_loop(0, rem, step, 0)
            cpo = pltpu.make_async_copy(
                ych_v, out_ref.at[pl.ds(rows0, CT * NSEQ), :], wsem)
            cpo.start()
            cpo.wait()
            return carry

        lax.fori_loop(0, nch, chunk, 0)

    gru_pass(xpad, None, w0f, wh0f, b0f, bh0f, y0f, False, True)
    gru_pass(xpad, None, w0b, wh0b, b0b, bh0b, y0b, True, True)
    gru_pass(y0f, y0b, w1f, wh1f, b1f, bh1f, y1f, False, False)
    gru_pass(y0f, y0b, w1b, wh1b, b1b, bh1b, y1b, True, False)


def _gru(lmax, lens, xpad, weights):
    yshape = jax.ShapeDtypeStruct((PACK_ROWS, HGRU), jnp.float32)
    vspec = pl.BlockSpec(memory_space=pltpu.VMEM)
    return pl.pallas_call(
        _gru_body,
        in_specs=[pl.BlockSpec(memory_space=pltpu.SMEM),
                  vspec,
                  pl.BlockSpec(memory_space=pl.ANY)]
        + [vspec] * 16,
        out_specs=[pl.BlockSpec(memory_space=pl.ANY)] * 4,
        out_shape=[yshape] * 4,
        scratch_shapes=[
            pltpu.VMEM((CT * NSEQ, D), jnp.float32),
            pltpu.VMEM((CT * NSEQ, HGRU), jnp.float32),
            pltpu.VMEM((CT * NSEQ, HGRU), jnp.float32),
            pltpu.VMEM((CT * NSEQ, HGRU), jnp.float32),
            pltpu.VMEM((NSEQ, HGRU), jnp.float32),
            pltpu.SemaphoreType.DMA,
            pltpu.SemaphoreType.DMA,
        ],
    )(lmax, lens, xpad, *weights)


def _head_body(xf_ref, xb_ref, xg_ref, rg_ref, rb_ref, w1T_ref, b1_ref,
               mg_ref, mb_ref, w2T_ref, b2_ref, waT_ref, wbT_ref, bc_ref,
               o_ref):
    u = jnp.concatenate([xf_ref[...], xb_ref[...]], axis=1)
    m = jnp.mean(u, axis=-1, keepdims=True)
    v = jnp.mean((u - m) ** 2, axis=-1, keepdims=True)
    u = (u - m) / jnp.sqrt(v + 1e-5) * rg_ref[...] + rb_ref[...]
    u = jax.nn.relu(
        jnp.dot(u, w1T_ref[...], preferred_element_type=jnp.float32)
        + b1_ref[...])
    m = jnp.mean(u, axis=-1, keepdims=True)
    v = jnp.mean((u - m) ** 2, axis=-1, keepdims=True)
    u = (u - m) / jnp.sqrt(v + 1e-5) * mg_ref[...] + mb_ref[...]
    u = jnp.dot(u, w2T_ref[...], preferred_element_type=jnp.float32) \
        + b2_ref[...]
    o_ref[...] = (jnp.dot(u, waT_ref[...], preferred_element_type=jnp.float32)
                  + jnp.dot(xg_ref[...], wbT_ref[...],
                            preferred_element_type=jnp.float32)
                  + bc_ref[...])


def _head(xf, xb, xg, rg, rb, w1T, b1, mg, mb, w2T, b2, waT, wbT, bc):
    row = pl.BlockSpec((1024, D), lambda i: (i, 0))
    half = pl.BlockSpec((1024, HGRU), lambda i: (i, 0))
    wfull = pl.BlockSpec((D, D), lambda i: (0, 0))
    bfull = pl.BlockSpec((1, D), lambda i: (0, 0))
    return pl.pallas_call(
        _head_body,
        grid=(2,),
        in_specs=[half, half, row, bfull, bfull, wfull, bfull, bfull, bfull,
                  wfull, bfull, wfull, wfull, bfull],
        out_specs=row,
        out_shape=jax.ShapeDtypeStruct((BS, D), jnp.float32),
    )(xf, xb, xg, rg, rb, w1T, b1, mg, mb, w2T, b2, waT, wbT, bc)


def kernel(x, edge_index, batch, batch_size, neighbor_mask_node,
           neighbor_mask_edge, W_proj, b_proj, Wl0, bl0, Wr0, Wl1, bl1, Wr1,
           ln0_g, ln0_b, Wih0f, Whh0f, bih0f, bhh0f, Wih0b, Whh0b, bih0b,
           bhh0b, Wih1f, Whh1f, bih1f, bhh1f, Wih1b, Whh1b, bih1b, bhh1b,
           rnn_g, rnn_b, Wm1, bm1, mlp_g, mlp_b, Wm2, bm2, Wcat, bcat):
    f32 = jnp.float32
    x = x.astype(f32)
    bs = batch.shape[0]
    batch = batch.astype(jnp.int32)

    lengths = jnp.bincount(batch, length=NSEQ).astype(jnp.int32)
    starts = jnp.concatenate(
        [jnp.zeros((1,), jnp.int32), jnp.cumsum(lengths)[:-1].astype(jnp.int32)])
    pos = jnp.arange(bs, dtype=jnp.int32) - starts[batch]
    ridx = pos * NSEQ + batch
    lmax = jnp.max(lengths).reshape(1)
    sel = jnp.arange(bs) < batch_size
    x_t = jnp.where(sel[:, None], x[:bs], 0.0)

    xpad = _sc_pack(x_t, ridx)
    gw = (Wih0f.T, Whh0f.T, (bih0f).reshape(1, -1), (bhh0f).reshape(1, -1),
          Wih0b.T, Whh0b.T, (bih0b).reshape(1, -1), (bhh0b).reshape(1, -1),
          Wih1f.T, Whh1f.T, (bih1f).reshape(1, -1), (bhh1f).reshape(1, -1),
          Wih1b.T, Whh1b.T, (bih1b).reshape(1, -1), (bhh1b).reshape(1, -1))
    y0f, y0b, y1f, y1b = _gru(lmax.astype(jnp.int32),
                              lengths.reshape(NSEQ, 1), xpad,
                              [w.astype(f32) for w in gw])
    xrf, xrb = _sc_unpack(y1f, y1b, ridx)

    src = edge_index[0].astype(jnp.int32)
    dst = edge_index[1].astype(jnp.int32)
    epad = NS * EPS
    src_p = jnp.concatenate(
        [src, jnp.zeros((epad - src.shape[0],), jnp.int32)])
    dst_p = jnp.concatenate(
        [dst, jnp.full((epad - dst.shape[0],), N_NODES, jnp.int32)])
    src_r = src_p.reshape(NS, NCHUNK, ECH)
    src_p = jnp.stack([src_r, src_r + N_NODES])
    dst_p = dst_p.reshape(NS, NCHUNK, ECH)
    zrow = jnp.zeros((ECH, HD), f32)
    zc = jnp.zeros((ECH, 16), f32)
    ones = jnp.ones((ECH, 16), f32)

    def halves(h):
        return jnp.concatenate([h[:, :HD], h[:, HD:]], axis=0)

    h0 = _proj(x, W_proj.T.astype(f32), b_proj.reshape(1, -1).astype(f32))
    s0, cnt = _sc_edges_cnt(halves(h0), src_p, dst_p, zrow, zc, ones)
    h1 = _combine0(s0, cnt, x, Wl0.T.astype(f32), bl0.reshape(1, -1),
                   Wr0.T.astype(f32), ln0_g.reshape(1, -1),
                   ln0_b.reshape(1, -1))
    s1 = _sc_edges(halves(h1), src_p, dst_p, zrow)
    x_gnn = _combine1(s1, cnt, h1[:BS], Wl1.T.astype(f32), bl1.reshape(1, -1),
                      Wr1.T.astype(f32))
    x_gnn = jnp.where(sel[:, None], x_gnn, 0.0)

    waT = Wcat[:, :D].T.astype(f32)
    wbT = Wcat[:, D:].T.astype(f32)
    return _head(xrf, xrb, x_gnn, rnn_g.reshape(1, -1), rnn_b.reshape(1, -1),
                 Wm1.T.astype(f32), bm1.reshape(1, -1), mlp_g.reshape(1, -1),
                 mlp_b.reshape(1, -1), Wm2.T.astype(f32), bm2.reshape(1, -1),
                 waT, wbT, bcat.reshape(1, -1))


# speedup vs baseline: 34.6165x; 1.0201x over previous
import functools

import jax
import jax.numpy as jnp
from jax import lax
from jax.experimental import pallas as pl
from jax.experimental.pallas import tpu as pltpu
from jax.experimental.pallas import tpu_sc as plsc

NSEQ = 64
BS = 2048
N_NODES = 10000
D = 128
HGRU = 64
E_EDGES = 320000

NC, NS = 2, 16
NW = NC * NS
HD = D // 2
EPS = 20480
ECH = 128
NCHUNK = EPS // ECH
ACC_ROWS = 10240
ROWS_PW = ACC_ROWS // NS
PACK_ROWS = BS * NSEQ

_mesh = plsc.VectorSubcoreMesh(core_axis_name="c", subcore_axis_name="s")


def _wid():
    return lax.axis_index("s") * NC + lax.axis_index("c")


@functools.partial(
    pl.kernel,
    mesh=_mesh,
    out_type=jax.ShapeDtypeStruct((PACK_ROWS, D), jnp.float32),
    scratch_types=[
        pltpu.VMEM((BS // NW,), jnp.int32),
        pltpu.VMEM((BS // NW, D), jnp.float32),
        pltpu.SemaphoreType.DMA,
    ],
)
def _sc_pack(x_hbm, ridx_hbm, out_hbm, idx_v, rows_v, sem):
    w = _wid()
    base = w * (BS // NW)
    pltpu.sync_copy(ridx_hbm.at[pl.ds(base, BS // NW)], idx_v)
    pltpu.sync_copy(x_hbm.at[pl.ds(base, BS // NW)], rows_v)
    pltpu.async_copy(rows_v, out_hbm.at[idx_v], sem).wait()


@functools.partial(
    pl.kernel,
    mesh=_mesh,
    out_type=(
        jax.ShapeDtypeStruct((BS, HGRU), jnp.float32),
        jax.ShapeDtypeStruct((BS, HGRU), jnp.float32),
    ),
    scratch_types=[
        pltpu.VMEM((BS // NW,), jnp.int32),
        pltpu.VMEM((BS // NW, HGRU), jnp.float32),
        pltpu.VMEM((BS // NW, HGRU), jnp.float32),
        pltpu.SemaphoreType.DMA,
        pltpu.SemaphoreType.DMA,
    ],
    compiler_params=pltpu.CompilerParams(use_tc_tiling_on_sc=False),
)
def _sc_unpack(yf_hbm, yb_hbm, gidx_hbm, of_hbm, ob_hbm, idx_v, gf_v, gb_v,
               semf, semb):
    w = _wid()
    k = BS // NW
    base = w * k
    pltpu.sync_copy(gidx_hbm.at[pl.ds(base, k)], idx_v)
    cf = pltpu.async_copy(yf_hbm.at[idx_v], gf_v, semf)
    cb = pltpu.async_copy(yb_hbm.at[idx_v], gb_v, semb)
    cf.wait()
    cb.wait()
    pltpu.sync_copy(gf_v, of_hbm.at[pl.ds(base, k)])
    pltpu.sync_copy(gb_v, ob_hbm.at[pl.ds(base, k)])


def _edge_body(with_cnt, h_hbm, src_hbm, dst_hbm, zrow_hbm, zc_hbm, ones_hbm,
               acc_out, cnt_out, sidx_v, didx_v, rows_v, ones_v, acc_sh,
               cnt_sh, gsem0, gsem1, gsem2, gsem3, ssem0, ssem1, ssem2,
               ssem3):
    c = lax.axis_index("c")
    s = lax.axis_index("s")

    pltpu.sync_copy(zrow_hbm, rows_v.at[0])
    if with_cnt:
        pltpu.sync_copy(zc_hbm, ones_v)
    for j in range(ROWS_PW // ECH):
        pltpu.sync_copy(rows_v.at[0],
                        acc_sh.at[pl.ds(s * ROWS_PW + j * ECH, ECH)])
        if with_cnt:
            pltpu.sync_copy(ones_v,
                            cnt_sh.at[pl.ds(s * ROWS_PW + j * ECH, ECH)])
    if with_cnt:
        pltpu.sync_copy(ones_hbm, ones_v)
    pltpu.sync_copy(src_hbm.at[c, s], sidx_v)
    pltpu.sync_copy(dst_hbm.at[s], didx_v)
    plsc.subcore_barrier()

    gsems = (gsem0, gsem1, gsem2, gsem3)
    ssems = (ssem0, ssem1, ssem2, ssem3)
    pltpu.async_copy(h_hbm.at[sidx_v.at[0]], rows_v.at[0], gsems[0])
    pltpu.async_copy(h_hbm.at[sidx_v.at[1]], rows_v.at[1], gsems[1])

    def quad(i, carry):
        for b in range(4):
            j = 4 * i + b
            pltpu.make_async_copy(h_hbm.at[sidx_v.at[j]], rows_v.at[b],
                                  gsems[b]).wait()
            pltpu.async_copy(rows_v.at[b], acc_sh.at[didx_v.at[j]], ssems[b],
                             add=True)
            if with_cnt:
                pltpu.sync_copy(ones_v, cnt_sh.at[didx_v.at[j]], add=True)
            bg = (b + 2) % 4

            @pl.when(j >= 2)
            def _():
                pltpu.make_async_copy(rows_v.at[bg],
                                      acc_sh.at[didx_v.at[0]],
                                      ssems[bg]).wait()

            @pl.when(j + 2 < NCHUNK)
            def _():
                pltpu.async_copy(h_hbm.at[sidx_v.at[j + 2]], rows_v.at[bg],
                                 gsems[bg])
        return carry

    lax.fori_loop(0, NCHUNK // 4, quad, 0)
    for b in ((NCHUNK - 2) % 4, (NCHUNK - 1) % 4):
        pltpu.make_async_copy(rows_v.at[b], acc_sh.at[didx_v.at[0]],
                              ssems[b]).wait()
    plsc.subcore_barrier()

    rbase = s * ROWS_PW
    pltpu.sync_copy(acc_sh.at[pl.ds(rbase, ROWS_PW)],
                    acc_out.at[c, pl.ds(rbase, ROWS_PW)])
    if with_cnt:
        pltpu.sync_copy(cnt_sh.at[pl.ds(rbase, ROWS_PW)],
                        cnt_out.at[c, pl.ds(rbase, ROWS_PW)])


@functools.partial(
    pl.kernel,
    mesh=_mesh,
    out_type=(
        jax.ShapeDtypeStruct((NC, ACC_ROWS, HD), jnp.float32),
        jax.ShapeDtypeStruct((NC, ACC_ROWS, 16), jnp.float32),
    ),
    scratch_types=[
        pltpu.VMEM((NCHUNK, ECH), jnp.int32),
        pltpu.VMEM((NCHUNK, ECH), jnp.int32),
        pltpu.VMEM((4, ECH, HD), jnp.float32),
        pltpu.VMEM((ECH, 16), jnp.float32),
        pltpu.VMEM_SHARED((ACC_ROWS, HD), jnp.float32),
        pltpu.VMEM_SHARED((ACC_ROWS, 16), jnp.float32),
    ] + [pltpu.SemaphoreType.DMA] * 8,
    compiler_params=pltpu.CompilerParams(use_tc_tiling_on_sc=False),
)
def _sc_edges_cnt(h_hbm, src_hbm, dst_hbm, zrow_hbm, zc_hbm, ones_hbm,
                  acc_out, cnt_out, sidx_v, didx_v, rows_v, ones_v, acc_sh,
                  cnt_sh, *sems):
    _edge_body(True, h_hbm, src_hbm, dst_hbm, zrow_hbm, zc_hbm, ones_hbm,
               acc_out, cnt_out, sidx_v, didx_v, rows_v, ones_v, acc_sh,
               cnt_sh, *sems)


@functools.partial(
    pl.kernel,
    mesh=_mesh,
    out_type=jax.ShapeDtypeStruct((NC, ACC_ROWS, HD), jnp.float32),
    scratch_types=[
        pltpu.VMEM((NCHUNK, ECH), jnp.int32),
        pltpu.VMEM((NCHUNK, ECH), jnp.int32),
        pltpu.VMEM((4, ECH, HD), jnp.float32),
        pltpu.VMEM_SHARED((ACC_ROWS, HD), jnp.float32),
    ] + [pltpu.SemaphoreType.DMA] * 8,
    compiler_params=pltpu.CompilerParams(use_tc_tiling_on_sc=False),
)
def _sc_edges(h_hbm, src_hbm, dst_hbm, zrow_hbm, acc_out, sidx_v, didx_v,
              rows_v, acc_sh, *sems):
    _edge_body(False, h_hbm, src_hbm, dst_hbm, zrow_hbm, None, None,
               acc_out, None, sidx_v, didx_v, rows_v, None, acc_sh, None,
               *sems)


def _proj_body(x_ref, w_ref, b_ref, o_ref):
    o_ref[...] = jax.nn.relu(
        jnp.dot(x_ref[...], w_ref[...], preferred_element_type=jnp.float32)
        + b_ref[...])


def _proj(x, wT, b):
    return pl.pallas_call(
        _proj_body,
        grid=(10,),
        in_specs=[
            pl.BlockSpec((1000, D), lambda i: (i, 0)),
            pl.BlockSpec((D, D), lambda i: (0, 0)),
            pl.BlockSpec((1, D), lambda i: (0, 0)),
        ],
        out_specs=pl.BlockSpec((1000, D), lambda i: (i, 0)),
        out_shape=jax.ShapeDtypeStruct((N_NODES, D), jnp.float32),
    )(x, wT, b)


def _comb0_body(s_ref, c_ref, x_ref, wlT_ref, bl_ref, wrT_ref, g_ref, be_ref,
                o_ref):
    ssum = jnp.concatenate([s_ref[0], s_ref[1]], axis=-1)
    cnt = c_ref[0, :, 0:1]
    agg = ssum / jnp.maximum(cnt, 1.0)
    out = (jnp.dot(agg, wlT_ref[...], preferred_element_type=jnp.float32)
           + bl_ref[...]
           + jnp.dot(x_ref[...], wrT_ref[...],
                     preferred_element_type=jnp.float32))
    nrm = jnp.maximum(
        jnp.sqrt(jnp.sum(out * out, axis=-1, keepdims=True)), 1e-12)
    out = jax.nn.relu(out / nrm)
    m = jnp.mean(out, axis=-1, keepdims=True)
    v = jnp.mean((out - m) ** 2, axis=-1, keepdims=True)
    o_ref[...] = (out - m) / jnp.sqrt(v + 1e-5) * g_ref[...] + be_ref[...]


def _combine0(sparts, cparts, x, wlT, bl, wrT, g, be):
    return pl.pallas_call(
        _comb0_body,
        grid=(10,),
        in_specs=[
            pl.BlockSpec((NC, 1000, HD), lambda i: (0, i, 0)),
            pl.BlockSpec((1, 1000, 16), lambda i: (0, i, 0)),
            pl.BlockSpec((1000, D), lambda i: (i, 0)),
            pl.BlockSpec((D, D), lambda i: (0, 0)),
            pl.BlockSpec((1, D), lambda i: (0, 0)),
            pl.BlockSpec((D, D), lambda i: (0, 0)),
            pl.BlockSpec((1, D), lambda i: (0, 0)),
            pl.BlockSpec((1, D), lambda i: (0, 0)),
        ],
        out_specs=pl.BlockSpec((1000, D), lambda i: (i, 0)),
        out_shape=jax.ShapeDtypeStruct((N_NODES, D), jnp.float32),
    )(sparts, cparts, x, wlT, bl, wrT, g, be)


def _comb1_body(s_ref, c_ref, h_ref, wlT_ref, bl_ref, wrT_ref, o_ref):
    ssum = jnp.concatenate([s_ref[0], s_ref[1]], axis=-1)
    cnt = c_ref[0, :, 0:1]
    agg = ssum / jnp.maximum(cnt, 1.0)
    o_ref[...] = (jnp.dot(agg, wlT_ref[...], preferred_element_type=jnp.float32)
                  + bl_ref[...]
                  + jnp.dot(h_ref[...], wrT_ref[...],
                            preferred_element_type=jnp.float32))


def _combine1(sparts, cparts, h1, wlT, bl, wrT):
    return pl.pallas_call(
        _comb1_body,
        grid=(2,),
        in_specs=[
            pl.BlockSpec((NC, 1024, HD), lambda i: (0, i, 0)),
            pl.BlockSpec((1, 1024, 16), lambda i: (0, i, 0)),
            pl.BlockSpec((1024, D), lambda i: (i, 0)),
            pl.BlockSpec((D, D), lambda i: (0, 0)),
            pl.BlockSpec((1, D), lambda i: (0, 0)),
            pl.BlockSpec((D, D), lambda i: (0, 0)),
        ],
        out_specs=pl.BlockSpec((1024, D), lambda i: (i, 0)),
        out_shape=jax.ShapeDtypeStruct((BS, D), jnp.float32),
    )(sparts, cparts, h1, wlT, bl, wrT)


CT = 64


def _gru_body(lmax_ref, lens_ref, xpad, w0f, wh0f, b0f, bh0f, w0b, wh0b, b0b,
              bh0b, w1f, wh1f, b1f, bh1f, w1b, wh1b, b1b, bh1b,
              y0f, y0b, y1f, y1b, xch_v, af_v, ab_v, ych_v, h_v, sem, wsem):
    lmax = lmax_ref[0]
    lens = lens_ref[...]
    nch = (lmax + CT - 1) // CT

    def gru_pass(in_f, in_b, wi, wh, bi, bh, out_ref, reverse, mask_inputs):
        h_v[...] = jnp.zeros((NSEQ, HGRU), jnp.float32)

        def chunk(ci, carry):
            cc = (nch - 1 - ci) if reverse else ci
            base_t = cc * CT
            rem = jnp.minimum(lmax - base_t, CT)
            rows0 = pl.multiple_of(base_t * NSEQ, NSEQ)
            if in_b is None:
                cpf = pltpu.make_async_copy(
                    in_f.at[pl.ds(rows0, CT * NSEQ), :], xch_v, sem)
                cpf.start()
                cpf.wait()
            else:
                cpf = pltpu.make_async_copy(
                    in_f.at[pl.ds(rows0, CT * NSEQ), :], af_v, sem)
                cpb = pltpu.make_async_copy(
                    in_b.at[pl.ds(rows0, CT * NSEQ), :], ab_v, sem)
                cpf.start()
                cpb.start()
                cpf.wait()
                cpb.wait()

            def step(tt, c2):
                si = (rem - 1 - tt) if reverse else tt
                r0 = si * NSEQ
                if in_b is None:
                    xt = xch_v[pl.ds(r0, NSEQ), :]
                else:
                    xt = jnp.concatenate(
                        [af_v[pl.ds(r0, NSEQ), :], ab_v[pl.ds(r0, NSEQ), :]],
                        axis=1)
                if mask_inputs:
                    xt = jnp.where(lens > base_t + si, xt, 0.0)
                h = h_v[...]
                gi = jnp.dot(xt, wi[...],
                             preferred_element_type=jnp.float32) + bi[...]
                gh = jnp.dot(h, wh[...],
                             preferred_element_type=jnp.float32) + bh[...]
                r = jax.nn.sigmoid(gi[:, :HGRU] + gh[:, :HGRU])
                z = jax.nn.sigmoid(gi[:, HGRU:2 * HGRU]
                                   + gh[:, HGRU:2 * HGRU])
                n = jnp.tanh(gi[:, 2 * HGRU:] + r * gh[:, 2 * HGRU:])
                hn = (1.0 - z) * n + z * h
                h_v[...] = hn
                ych_v[pl.ds(r0, NSEQ), :] = hn
                return c2

            lax.fori_loop(0, rem, step, 0)
            cpo = pltpu.make_async_copy(
                ych_v, out_ref.at[pl.ds(rows0, CT * NSEQ), :], wsem)
            cpo.start()
            cpo.wait()
            return carry

        lax.fori_loop(0, nch, chunk, 0)

    gru_pass(xpad, None, w0f, wh0f, b0f, bh0f, y0f, False, True)
    gru_pass(xpad, None, w0b, wh0b, b0b, bh0b, y0b, True, True)
    gru_pass(y0f, y0b, w1f, wh1f, b1f, bh1f, y1f, False, False)
    gru_pass(y0f, y0b, w1b, wh1b, b1b, bh1b, y1b, True, False)


def _gru(lmax, lens, xpad, weights):
    yshape = jax.ShapeDtypeStruct((PACK_ROWS, HGRU), jnp.float32)
    vspec = pl.BlockSpec(memory_space=pltpu.VMEM)
    return pl.pallas_call(
        _gru_body,
        in_specs=[pl.BlockSpec(memory_space=pltpu.SMEM),
                  vspec,
                  pl.BlockSpec(memory_space=pl.ANY)]
        + [vspec] * 16,
        out_specs=[pl.BlockSpec(memory_space=pl.ANY)] * 4,
        out_shape=[yshape] * 4,
        scratch_shapes=[
            pltpu.VMEM((CT * NSEQ, D), jnp.float32),
            pltpu.VMEM((CT * NSEQ, HGRU), jnp.float32),
            pltpu.VMEM((CT * NSEQ, HGRU), jnp.float32),
            pltpu.VMEM((CT * NSEQ, HGRU), jnp.float32),
            pltpu.VMEM((NSEQ, HGRU), jnp.float32),
            pltpu.SemaphoreType.DMA,
            pltpu.SemaphoreType.DMA,
        ],
    )(lmax, lens, xpad, *weights)


def _head_body(xf_ref, xb_ref, xg_ref, rg_ref, rb_ref, w1T_ref, b1_ref,
               mg_ref, mb_ref, w2T_ref, b2_ref, waT_ref, wbT_ref, bc_ref,
               o_ref):
    u = jnp.concatenate([xf_ref[...], xb_ref[...]], axis=1)
    m = jnp.mean(u, axis=-1, keepdims=True)
    v = jnp.mean((u - m) ** 2, axis=-1, keepdims=True)
    u = (u - m) / jnp.sqrt(v + 1e-5) * rg_ref[...] + rb_ref[...]
    u = jax.nn.relu(
        jnp.dot(u, w1T_ref[...], preferred_element_type=jnp.float32)
        + b1_ref[...])
    m = jnp.mean(u, axis=-1, keepdims=True)
    v = jnp.mean((u - m) ** 2, axis=-1, keepdims=True)
    u = (u - m) / jnp.sqrt(v + 1e-5) * mg_ref[...] + mb_ref[...]
    u = jnp.dot(u, w2T_ref[...], preferred_element_type=jnp.float32) \
        + b2_ref[...]
    o_ref[...] = (jnp.dot(u, waT_ref[...], preferred_element_type=jnp.float32)
                  + jnp.dot(xg_ref[...], wbT_ref[...],
                            preferred_element_type=jnp.float32)
                  + bc_ref[...])


def _head(xf, xb, xg, rg, rb, w1T, b1, mg, mb, w2T, b2, waT, wbT, bc):
    row = pl.BlockSpec((1024, D), lambda i: (i, 0))
    half = pl.BlockSpec((1024, HGRU), lambda i: (i, 0))
    wfull = pl.BlockSpec((D, D), lambda i: (0, 0))
    bfull = pl.BlockSpec((1, D), lambda i: (0, 0))
    return pl.pallas_call(
        _head_body,
        grid=(2,),
        in_specs=[half, half, row, bfull, bfull, wfull, bfull, bfull, bfull,
                  wfull, bfull, wfull, wfull, bfull],
        out_specs=row,
        out_shape=jax.ShapeDtypeStruct((BS, D), jnp.float32),
    )(xf, xb, xg, rg, rb, w1T, b1, mg, mb, w2T, b2, waT, wbT, bc)


def kernel(x, edge_index, batch, batch_size, neighbor_mask_node,
           neighbor_mask_edge, W_proj, b_proj, Wl0, bl0, Wr0, Wl1, bl1, Wr1,
           ln0_g, ln0_b, Wih0f, Whh0f, bih0f, bhh0f, Wih0b, Whh0b, bih0b,
           bhh0b, Wih1f, Whh1f, bih1f, bhh1f, Wih1b, Whh1b, bih1b, bhh1b,
           rnn_g, rnn_b, Wm1, bm1, mlp_g, mlp_b, Wm2, bm2, Wcat, bcat):
    f32 = jnp.float32
    x = x.astype(f32)
    bs = batch.shape[0]
    batch = batch.astype(jnp.int32)

    lengths = jnp.bincount(batch, length=NSEQ).astype(jnp.int32)
    starts = jnp.concatenate(
        [jnp.zeros((1,), jnp.int32), jnp.cumsum(lengths)[:-1].astype(jnp.int32)])
    pos = jnp.arange(bs, dtype=jnp.int32) - starts[batch]
    ridx = pos * NSEQ + batch
    lmax = jnp.max(lengths).reshape(1)
    sel = jnp.arange(bs) < batch_size
    x_t = jnp.where(sel[:, None], x[:bs], 0.0)

    xpad = _sc_pack(x_t, ridx)
    gw = (Wih0f.T, Whh0f.T, (bih0f).reshape(1, -1), (bhh0f).reshape(1, -1),
          Wih0b.T, Whh0b.T, (bih0b).reshape(1, -1), (bhh0b).reshape(1, -1),
          Wih1f.T, Whh1f.T, (bih1f).reshape(1, -1), (bhh1f).reshape(1, -1),
          Wih1b.T, Whh1b.T, (bih1b).reshape(1, -1), (bhh1b).reshape(1, -1))
    y0f, y0b, y1f, y1b = _gru(lmax.astype(jnp.int32),
                              lengths.reshape(NSEQ, 1), xpad,
                              [w.astype(f32) for w in gw])
    xrf, xrb = _sc_unpack(y1f, y1b, ridx)

    src = edge_index[0].astype(jnp.int32)
    dst = edge_index[1].astype(jnp.int32)
    epad = NS * EPS
    src_p = jnp.concatenate(
        [src, jnp.zeros((epad - src.shape[0],), jnp.int32)])
    dst_p = jnp.concatenate(
        [dst, jnp.full((epad - dst.shape[0],), N_NODES, jnp.int32)])
    src_r = src_p.reshape(NS, NCHUNK, ECH)
    src_p = jnp.stack([src_r, src_r + N_NODES])
    dst_p = dst_p.reshape(NS, NCHUNK, ECH)
    zrow = jnp.zeros((ECH, HD), f32)
    zc = jnp.zeros((ECH, 16), f32)
    ones = jnp.ones((ECH, 16), f32)

    def halves(h):
        return jnp.concatenate([h[:, :HD], h[:, HD:]], axis=0)

    h0 = _proj(x, W_proj.T.astype(f32), b_proj.reshape(1, -1).astype(f32))
    s0, cnt = _sc_edges_cnt(halves(h0), src_p, dst_p, zrow, zc, ones)
    h1 = _combine0(s0, cnt, x, Wl0.T.astype(f32), bl0.reshape(1, -1),
                   Wr0.T.astype(f32), ln0_g.reshape(1, -1),
                   ln0_b.reshape(1, -1))
    s1 = _sc_edges(halves(h1), src_p, dst_p, zrow)
    x_gnn = _combine1(s1, cnt, h1[:BS], Wl1.T.astype(f32), bl1.reshape(1, -1),
                      Wr1.T.astype(f32))
    x_gnn = jnp.where(sel[:, None], x_gnn, 0.0)

    waT = Wcat[:, :D].T.astype(f32)
    wbT = Wcat[:, D:].T.astype(f32)
    return _head(xrf, xrb, x_gnn, rnn_g.reshape(1, -1), rnn_b.reshape(1, -1),
                 Wm1.T.astype(f32), bm1.reshape(1, -1), mlp_g.reshape(1, -1),
                 mlp_b.reshape(1, -1), Wm2.T.astype(f32), bm2.reshape(1, -1),
                 waT, wbT, bcat.reshape(1, -1))
